# baseline (speedup 1.0000x reference)
import functools

import jax
import jax.numpy as jnp
from jax import lax
from jax.experimental import pallas as pl
from jax.experimental.pallas import tpu as pltpu
from jax.experimental.pallas import tpu_sc as plsc

EMB = 128
BATCH = 16384
NC = 2
NS = 16
NW = NC * NS
LANES = 16

_mesh = plsc.VectorSubcoreMesh(core_axis_name="c", subcore_axis_name="s")


def _make_sc_diffs(rows, chunk):
    bpw = rows // NW
    nch = bpw // chunk
    out_rows = rows * LANES // EMB
    orpw = out_rows // NW
    orpc = orpw // nch

    @functools.partial(
        pl.kernel,
        mesh=_mesh,
        out_type=jax.ShapeDtypeStruct((out_rows, EMB), jnp.float32),
        scratch_types=[
            pltpu.VMEM((bpw,), jnp.int32),
            pltpu.VMEM((bpw,), jnp.int32),
            pltpu.VMEM((bpw,), jnp.int32),
            pltpu.VMEM((chunk, EMB), jnp.float32),
            pltpu.VMEM((chunk, EMB), jnp.float32),
            pltpu.VMEM((chunk, EMB), jnp.float32),
            pltpu.VMEM((chunk, EMB), jnp.float32),
            pltpu.VMEM((chunk, EMB), jnp.float32),
            pltpu.VMEM((chunk, EMB), jnp.float32),
            pltpu.VMEM((orpc, EMB), jnp.float32),
            pltpu.SemaphoreType.DMA,
            pltpu.SemaphoreType.DMA,
        ],
    )
    def _sc_diffs(ut, it, uix, pix, nix, out, uidx, pidx, nidx,
                  ub0, pb0, nb0, ub1, pb1, nb1, ov, sem0, sem1):
        wid = lax.axis_index("s") * NC + lax.axis_index("c")
        base = pl.multiple_of(wid * bpw, 8)
        pltpu.sync_copy(uix.at[pl.ds(base, bpw)], uidx)
        pltpu.sync_copy(pix.at[pl.ds(base, bpw)], pidx)
        pltpu.sync_copy(nix.at[pl.ds(base, bpw)], nidx)
        bufs = ((ub0, pb0, nb0, sem0), (ub1, pb1, nb1, sem1))

        def start(j):
            ub, pb, nb, sem = bufs[j % 2]
            sl = pl.ds(j * chunk, chunk)
            return (pltpu.async_copy(ut.at[uidx.at[sl]], ub, sem),
                    pltpu.async_copy(it.at[pidx.at[sl]], pb, sem),
                    pltpu.async_copy(it.at[nidx.at[sl]], nb, sem))

        pend = start(0)
        for j in range(nch):
            nxt = start(j + 1) if j + 1 < nch else None
            for cpy in pend:
                cpy.wait()
            ub, pb, nb, _ = bufs[j % 2]

            def row8_body(o, _, ub=ub, pb=pb, nb=nb):
                for i in range(8):
                    r = o * 8 + i
                    acc = jnp.zeros((LANES,), jnp.float32)
                    for k in range(EMB // LANES):
                        u = ub[r, pl.ds(k * LANES, LANES)]
                        p = pb[r, pl.ds(k * LANES, LANES)]
                        n = nb[r, pl.ds(k * LANES, LANES)]
                        acc = acc + u * (n - p)
                    ov[o, pl.ds(i * LANES, LANES)] = acc
                return 0

            lax.fori_loop(0, orpc, row8_body, 0)
            obase = pl.multiple_of(wid * orpw + j * orpc, 8)
            pltpu.sync_copy(ov, out.at[pl.ds(obase, orpc)])
            pend = nxt

    return _sc_diffs


HALF = BATCH // 2
_sc_half = _make_sc_diffs(HALF, 128)


def _softplus_mean_body(x0_ref, x1_ref, o_ref):
    row = lax.broadcasted_iota(jnp.int32, (EMB, 8), 0)
    col = lax.broadcasted_iota(jnp.int32, (EMB, 8), 1)
    sel = (row // LANES == col).astype(jnp.float32)
    total = jnp.float32(0.0)
    for ref in (x0_ref, x1_ref):
        d = jax.lax.dot_general(ref[...], sel, (((1,), (0,)), ((), ())),
                                preferred_element_type=jnp.float32)
        sp = jnp.maximum(d, 0.0) + jnp.log1p(jnp.exp(-jnp.abs(d)))
        total = total + jnp.sum(sp)
    o_ref[0, 0] = total * (1.0 / BATCH)


_tc_reduce = pl.pallas_call(
    _softplus_mean_body,
    out_shape=jax.ShapeDtypeStruct((1, 1), jnp.float32),
    in_specs=[pl.BlockSpec(memory_space=pltpu.VMEM),
              pl.BlockSpec(memory_space=pltpu.VMEM)],
    out_specs=pl.BlockSpec(memory_space=pltpu.SMEM),
)


def kernel(user_table, item_table, users, pos, neg):
    u = users.astype(jnp.int32)
    p = pos.astype(jnp.int32)
    n = neg.astype(jnp.int32)
    part0 = _sc_half(user_table, item_table, u[:HALF], p[:HALF], n[:HALF])
    part1 = _sc_half(user_table, item_table, u[HALF:], p[HALF:], n[HALF:])
    return _tc_reduce(part0, part1)[0, 0]


# speedup vs baseline: 1.2132x; 1.2132x over previous
import functools

import jax
import jax.numpy as jnp
from jax import lax
from jax.experimental import pallas as pl
from jax.experimental.pallas import tpu as pltpu
from jax.experimental.pallas import tpu_sc as plsc

EMB = 128
BATCH = 16384
NC = 2
NS = 16
NW = NC * NS
BPW = BATCH // NW
C = 64
NCH = BPW // C
LANES = 16

OUT_ROWS = BATCH * LANES // EMB
ORPW = OUT_ROWS // NW
ORPC = ORPW // NCH

_mesh = plsc.VectorSubcoreMesh(core_axis_name="c", subcore_axis_name="s")


@functools.partial(
    pl.kernel,
    mesh=_mesh,
    out_type=jax.ShapeDtypeStruct((OUT_ROWS, EMB), jnp.float32),
    scratch_types=[
        pltpu.VMEM((BPW,), jnp.int32),
        pltpu.VMEM((BPW,), jnp.int32),
        pltpu.VMEM((BPW,), jnp.int32),
        pltpu.VMEM((C, EMB), jnp.float32),
        pltpu.VMEM((C, EMB), jnp.float32),
        pltpu.VMEM((C, EMB), jnp.float32),
        pltpu.VMEM((C, EMB), jnp.float32),
        pltpu.VMEM((C, EMB), jnp.float32),
        pltpu.VMEM((C, EMB), jnp.float32),
        pltpu.VMEM((ORPC, EMB), jnp.float32),
        pltpu.SemaphoreType.DMA,
        pltpu.SemaphoreType.DMA,
    ],
)
def _sc_diffs(ut, it, uix, pix, nix, out, uidx, pidx, nidx,
              ub0, pb0, nb0, ub1, pb1, nb1, ov, sem0, sem1):
    wid = lax.axis_index("s") * NC + lax.axis_index("c")
    base = pl.multiple_of(wid * BPW, 8)
    ix0 = pltpu.async_copy(uix.at[pl.ds(base, BPW)], uidx, sem0)
    ix1 = pltpu.async_copy(pix.at[pl.ds(base, BPW)], pidx, sem0)
    ix2 = pltpu.async_copy(nix.at[pl.ds(base, BPW)], nidx, sem0)
    ix0.wait()
    ix1.wait()
    ix2.wait()
    bufs = ((ub0, pb0, nb0, sem0), (ub1, pb1, nb1, sem1))

    def start(j):
        ub, pb, nb, sem = bufs[j % 2]
        sl = pl.ds(j * C, C)
        return (pltpu.async_copy(ut.at[uidx.at[sl]], ub, sem),
                pltpu.async_copy(it.at[pidx.at[sl]], pb, sem),
                pltpu.async_copy(it.at[nidx.at[sl]], nb, sem))

    pend = start(0)
    for j in range(NCH):
        nxt = start(j + 1) if j + 1 < NCH else None
        for cpy in pend:
            cpy.wait()
        ub, pb, nb, _ = bufs[j % 2]

        def row8_body(o, _, ub=ub, pb=pb, nb=nb):
            for i in range(8):
                r = o * 8 + i
                acc = jnp.zeros((LANES,), jnp.float32)
                for k in range(EMB // LANES):
                    u = ub[r, pl.ds(k * LANES, LANES)]
                    p = pb[r, pl.ds(k * LANES, LANES)]
                    n = nb[r, pl.ds(k * LANES, LANES)]
                    acc = acc + u * (n - p)
                ov[o, pl.ds(i * LANES, LANES)] = acc
            return 0

        lax.fori_loop(0, ORPC, row8_body, 0)
        obase = pl.multiple_of(wid * ORPW + j * ORPC, 8)
        pltpu.sync_copy(ov, out.at[pl.ds(obase, ORPC)])
        pend = nxt


def _softplus_mean_body(x_ref, o_ref):
    x = x_ref[...]
    row = lax.broadcasted_iota(jnp.int32, (EMB, 8), 0)
    col = lax.broadcasted_iota(jnp.int32, (EMB, 8), 1)
    sel = (row // LANES == col).astype(jnp.float32)
    d = jax.lax.dot_general(x, sel, (((1,), (0,)), ((), ())),
                            preferred_element_type=jnp.float32)
    sp = jnp.maximum(d, 0.0) + jnp.log1p(jnp.exp(-jnp.abs(d)))
    o_ref[0, 0] = jnp.sum(sp) * (1.0 / BATCH)


_tc_reduce = pl.pallas_call(
    _softplus_mean_body,
    out_shape=jax.ShapeDtypeStruct((1, 1), jnp.float32),
    in_specs=[pl.BlockSpec(memory_space=pltpu.VMEM)],
    out_specs=pl.BlockSpec(memory_space=pltpu.SMEM),
)


def kernel(user_table, item_table, users, pos, neg):
    u = users.astype(jnp.int32)
    p = pos.astype(jnp.int32)
    n = neg.astype(jnp.int32)
    partials = _sc_diffs(user_table, item_table, u, p, n)
    return _tc_reduce(partials)[0, 0]


# speedup vs baseline: 1.2508x; 1.0310x over previous
import functools

import jax
import jax.numpy as jnp
from jax import lax
from jax.experimental import pallas as pl
from jax.experimental.pallas import tpu as pltpu
from jax.experimental.pallas import tpu_sc as plsc

EMB = 128
BATCH = 16384
NC = 2
NS = 16
NW = NC * NS
BPW = BATCH // NW
C = 128
NCH = BPW // C
LANES = 16

OUT_ROWS = BATCH * LANES // EMB
ORPW = OUT_ROWS // NW
ORPC = ORPW // NCH

_mesh = plsc.VectorSubcoreMesh(core_axis_name="c", subcore_axis_name="s")


@functools.partial(
    pl.kernel,
    mesh=_mesh,
    out_type=jax.ShapeDtypeStruct((OUT_ROWS, EMB), jnp.float32),
    scratch_types=[
        pltpu.VMEM((BPW,), jnp.int32),
        pltpu.VMEM((BPW,), jnp.int32),
        pltpu.VMEM((BPW,), jnp.int32),
        pltpu.VMEM((C, EMB), jnp.float32),
        pltpu.VMEM((C, EMB), jnp.float32),
        pltpu.VMEM((C, EMB), jnp.float32),
        pltpu.VMEM((C, EMB), jnp.float32),
        pltpu.VMEM((C, EMB), jnp.float32),
        pltpu.VMEM((C, EMB), jnp.float32),
        pltpu.VMEM((ORPC, EMB), jnp.float32),
        pltpu.SemaphoreType.DMA,
        pltpu.SemaphoreType.DMA,
    ],
)
def _sc_diffs(ut, it, uix, pix, nix, out, uidx, pidx, nidx,
              ub0, pb0, nb0, ub1, pb1, nb1, ov, sem0, sem1):
    wid = lax.axis_index("s") * NC + lax.axis_index("c")
    base = pl.multiple_of(wid * BPW, 8)
    ix0 = pltpu.async_copy(uix.at[pl.ds(base, BPW)], uidx, sem0)
    ix1 = pltpu.async_copy(pix.at[pl.ds(base, BPW)], pidx, sem0)
    ix2 = pltpu.async_copy(nix.at[pl.ds(base, BPW)], nidx, sem0)
    ix0.wait()
    ix1.wait()
    ix2.wait()
    bufs = ((ub0, pb0, nb0, sem0), (ub1, pb1, nb1, sem1))

    def start(j):
        ub, pb, nb, sem = bufs[j % 2]
        sl = pl.ds(j * C, C)
        return (pltpu.async_copy(ut.at[uidx.at[sl]], ub, sem),
                pltpu.async_copy(it.at[pidx.at[sl]], pb, sem),
                pltpu.async_copy(it.at[nidx.at[sl]], nb, sem))

    pend = start(0)
    for j in range(NCH):
        nxt = start(j + 1) if j + 1 < NCH else None
        for cpy in pend:
            cpy.wait()
        ub, pb, nb, _ = bufs[j % 2]

        def row8_body(o, _, ub=ub, pb=pb, nb=nb):
            for i in range(8):
                r = o * 8 + i
                acc = jnp.zeros((LANES,), jnp.float32)
                for k in range(EMB // LANES):
                    u = ub[r, pl.ds(k * LANES, LANES)]
                    p = pb[r, pl.ds(k * LANES, LANES)]
                    n = nb[r, pl.ds(k * LANES, LANES)]
                    acc = acc + u * (n - p)
                ov[o, pl.ds(i * LANES, LANES)] = acc
            return 0

        lax.fori_loop(0, ORPC, row8_body, 0)
        obase = pl.multiple_of(wid * ORPW + j * ORPC, 8)
        pltpu.sync_copy(ov, out.at[pl.ds(obase, ORPC)])
        pend = nxt


def _softplus_mean_body(x_ref, o_ref):
    x = x_ref[...]
    row = lax.broadcasted_iota(jnp.int32, (EMB, 8), 0)
    col = lax.broadcasted_iota(jnp.int32, (EMB, 8), 1)
    sel = (row // LANES == col).astype(jnp.float32)
    d = jax.lax.dot_general(x, sel, (((1,), (0,)), ((), ())),
                            preferred_element_type=jnp.float32)
    sp = jnp.maximum(d, 0.0) + jnp.log1p(jnp.exp(-jnp.abs(d)))
    o_ref[0, 0] = jnp.sum(sp) * (1.0 / BATCH)


_tc_reduce = pl.pallas_call(
    _softplus_mean_body,
    out_shape=jax.ShapeDtypeStruct((1, 1), jnp.float32),
    in_specs=[pl.BlockSpec(memory_space=pltpu.VMEM)],
    out_specs=pl.BlockSpec(memory_space=pltpu.SMEM),
)


def kernel(user_table, item_table, users, pos, neg):
    u = users.astype(jnp.int32)
    p = pos.astype(jnp.int32)
    n = neg.astype(jnp.int32)
    partials = _sc_diffs(user_table, item_table, u, p, n)
    return _tc_reduce(partials)[0, 0]
